# Initial kernel scaffold; baseline (speedup 1.0000x reference)
import functools

import jax
import jax.numpy as jnp
from jax import lax
from jax.experimental import pallas as pl
from jax.experimental.pallas import tpu as pltpu, tpu_sc as plsc

N_NODES = 10000
N_EDGES = 320000
D = 128

NC = 2
NS = 16
NW = NC * NS

CHUNK = 128
E_PER_W = 10240
E_PAD = NW * E_PER_W
CHUNKS_PER_W = E_PER_W // CHUNK
N_ACC = 10240
ROWS_PER_TEC = N_ACC // NS
DEG_W = 16


def _sc_aggregate(x, src2d, dst2d, zrows, zdeg, ones):
    mesh = plsc.VectorSubcoreMesh(
        core_axis_name="c", subcore_axis_name="s", num_cores=NC, num_subcores=NS
    )

    @functools.partial(
        pl.kernel,
        out_type=(
            jax.ShapeDtypeStruct((NC * N_ACC, D), jnp.float32),
            jax.ShapeDtypeStruct((NC * N_ACC, DEG_W), jnp.float32),
        ),
        mesh=mesh,
        scratch_types=[
            pltpu.VMEM_SHARED((N_ACC, D), jnp.float32),
            pltpu.VMEM_SHARED((N_ACC, DEG_W), jnp.float32),
            pltpu.VMEM((CHUNKS_PER_W, CHUNK), jnp.int32),
            pltpu.VMEM((CHUNKS_PER_W, CHUNK), jnp.int32),
            pltpu.VMEM((CHUNK, D), jnp.float32),
            pltpu.VMEM((CHUNK, DEG_W), jnp.float32),
            pltpu.SemaphoreType.DMA,
        ],
    )
    def agg_kernel(x_hbm, src_hbm, dst_hbm, zr_hbm, zd_hbm, ones_hbm,
                   agg_hbm, deg_hbm,
                   acc_sh, deg_sh, src_v, dst_v, rows_v, ones_v, sem):
        c = lax.axis_index("c")
        s = lax.axis_index("s")
        wid = c * NS + s

        pltpu.sync_copy(zr_hbm, acc_sh.at[pl.ds(s * ROWS_PER_TEC, ROWS_PER_TEC)])
        pltpu.sync_copy(zd_hbm, deg_sh.at[pl.ds(s * ROWS_PER_TEC, ROWS_PER_TEC)])
        pltpu.sync_copy(src_hbm.at[pl.ds(wid * CHUNKS_PER_W, CHUNKS_PER_W)], src_v)
        pltpu.sync_copy(dst_hbm.at[pl.ds(wid * CHUNKS_PER_W, CHUNKS_PER_W)], dst_v)
        pltpu.sync_copy(ones_hbm, ones_v)
        plsc.subcore_barrier()

        def chunk_body(j, carry):
            pltpu.async_copy(x_hbm.at[src_v.at[j]], rows_v, sem).wait()
            pltpu.sync_copy(rows_v, acc_sh.at[dst_v.at[j]], add=True)
            pltpu.sync_copy(ones_v, deg_sh.at[dst_v.at[j]], add=True)
            return carry

        lax.fori_loop(0, CHUNKS_PER_W, chunk_body, 0)
        plsc.subcore_barrier()

        out_base = c * N_ACC + s * ROWS_PER_TEC
        pltpu.sync_copy(acc_sh.at[pl.ds(s * ROWS_PER_TEC, ROWS_PER_TEC)],
                        agg_hbm.at[pl.ds(out_base, ROWS_PER_TEC)])
        pltpu.sync_copy(deg_sh.at[pl.ds(s * ROWS_PER_TEC, ROWS_PER_TEC)],
                        deg_hbm.at[pl.ds(out_base, ROWS_PER_TEC)])

    return agg_kernel(x, src2d, dst2d, zrows, zdeg, ones)


def _tc_combine(x, a0, a1, d0, d1, W_neigh, W_self, b2d, bias2d):
    R = 400
    grid = (N_NODES // R,)

    def body(x_ref, a0_ref, a1_ref, d0_ref, d1_ref, wn_ref, ws_ref,
             b_ref, bias_ref, o_ref):
        deg = d0_ref[:, :1] + d1_ref[:, :1]
        inv = 1.0 / jnp.maximum(deg, 1.0)
        mean = (a0_ref[...] + a1_ref[...]) * inv
        hn = lax.dot_general(mean, wn_ref[...], (((1,), (1,)), ((), ())),
                             preferred_element_type=jnp.float32)
        hs = lax.dot_general(x_ref[...], ws_ref[...], (((1,), (1,)), ((), ())),
                             preferred_element_type=jnp.float32)
        o_ref[...] = hs + hn + b_ref[...] + bias_ref[...]

    row_spec = pl.BlockSpec((R, D), lambda i: (i, 0))
    deg_spec = pl.BlockSpec((R, DEG_W), lambda i: (i, 0))
    full_spec = pl.BlockSpec((D, D), lambda i: (0, 0))
    vec_spec = pl.BlockSpec((1, D), lambda i: (0, 0))

    return pl.pallas_call(
        body,
        grid=grid,
        in_specs=[row_spec, row_spec, row_spec, deg_spec, deg_spec,
                  full_spec, full_spec, vec_spec, vec_spec],
        out_specs=row_spec,
        out_shape=jax.ShapeDtypeStruct((N_NODES, D), jnp.float32),
    )(x, a0, a1, d0, d1, W_neigh, W_self, b2d, bias2d)


def kernel(x, edge_index, W_neigh, W_self, b_self, bias):
    src = edge_index[0].astype(jnp.int32)
    dst = edge_index[1].astype(jnp.int32)
    n_pad = E_PAD - N_EDGES
    src2d = jnp.concatenate(
        [src, jnp.zeros((n_pad,), jnp.int32)]).reshape(NW * CHUNKS_PER_W, CHUNK)
    dst2d = jnp.concatenate(
        [dst, jnp.full((n_pad,), N_NODES, jnp.int32)]).reshape(NW * CHUNKS_PER_W, CHUNK)
    zrows = jnp.zeros((ROWS_PER_TEC, D), jnp.float32)
    zdeg = jnp.zeros((ROWS_PER_TEC, DEG_W), jnp.float32)
    ones = jnp.ones((CHUNK, DEG_W), jnp.float32)

    agg, deg = _sc_aggregate(x, src2d, dst2d, zrows, zdeg, ones)

    a0 = agg[:N_NODES]
    a1 = agg[N_ACC:N_ACC + N_NODES]
    d0 = deg[:N_NODES]
    d1 = deg[N_ACC:N_ACC + N_NODES]
    b2d = b_self.reshape(1, D)
    bias2d = bias.reshape(1, D)
    return _tc_combine(x, a0, a1, d0, d1, W_neigh, W_self, b2d, bias2d)


# speedup vs baseline: 3.0490x; 3.0490x over previous
import functools

import jax
import jax.numpy as jnp
from jax import lax
from jax.experimental import pallas as pl
from jax.experimental.pallas import tpu as pltpu, tpu_sc as plsc

N_NODES = 10000
N_EDGES = 320000
D = 128

NC = 2
NS = 16
NW = NC * NS

CHUNK = 128
E_PER_W = 10240
E_PAD = NW * E_PER_W
CHUNKS_PER_W = E_PER_W // CHUNK
N_ACC = 10112
ROWS_PER_TEC = N_ACC // NS
DEG_W = 16
STAGE = 16


def _sc_aggregate(x, src2d, dst2d, zrows):
    mesh = plsc.VectorSubcoreMesh(
        core_axis_name="c", subcore_axis_name="s", num_cores=NC, num_subcores=NS
    )

    @functools.partial(
        pl.kernel,
        out_type=jax.ShapeDtypeStruct((NC * N_ACC, D), jnp.float32),
        mesh=mesh,
        scratch_types=[
            pltpu.VMEM_SHARED((N_ACC, D), jnp.float32),
            pltpu.VMEM((CHUNK,), jnp.int32),
            pltpu.VMEM((CHUNK,), jnp.int32),
            pltpu.VMEM((CHUNK, D), jnp.float32),
            pltpu.SemaphoreType.DMA,
        ],
    )
    def agg_kernel(x_hbm, src_hbm, dst_hbm, zr_hbm,
                   agg_hbm,
                   acc_sh, src_v, dst_v, rows_v, sem):
        c = lax.axis_index("c")
        s = lax.axis_index("s")
        wid = c * NS + s
        acc_base = s * ROWS_PER_TEC
        out_base = c * N_ACC + s * ROWS_PER_TEC
        pieces = [(0, CHUNK), (CHUNK, CHUNK), (2 * CHUNK, CHUNK),
                  (3 * CHUNK, CHUNK), (4 * CHUNK, ROWS_PER_TEC - 4 * CHUNK)]

        pltpu.sync_copy(zr_hbm, rows_v)
        for off, sz in pieces:
            pltpu.sync_copy(rows_v.at[pl.ds(0, sz)],
                            acc_sh.at[pl.ds(acc_base + off, sz)])
        plsc.subcore_barrier()

        @pl.loop(0, CHUNKS_PER_W)
        def chunk_body(j):
            base = wid * E_PER_W + j * CHUNK
            pltpu.sync_copy(src_hbm.at[pl.ds(base, CHUNK)], src_v)
            pltpu.sync_copy(dst_hbm.at[pl.ds(base, CHUNK)], dst_v)
            pltpu.async_copy(x_hbm.at[src_v], rows_v, sem).wait()
            pltpu.sync_copy(rows_v, acc_sh.at[dst_v], add=True)

        plsc.subcore_barrier()
        for off, sz in pieces:
            pltpu.sync_copy(acc_sh.at[pl.ds(acc_base + off, sz)],
                            rows_v.at[pl.ds(0, sz)])
            pltpu.sync_copy(rows_v.at[pl.ds(0, sz)],
                            agg_hbm.at[pl.ds(out_base + off, sz)])

    return agg_kernel(x, src2d, dst2d, zrows)


def _sc_degree(dst1d, zrows, ones):
    mesh = plsc.VectorSubcoreMesh(
        core_axis_name="c", subcore_axis_name="s", num_cores=NC, num_subcores=NS
    )

    @functools.partial(
        pl.kernel,
        out_type=jax.ShapeDtypeStruct((NC * N_ACC, D), jnp.float32),
        mesh=mesh,
        scratch_types=[
            pltpu.VMEM_SHARED((N_ACC, D), jnp.float32),
            pltpu.VMEM((CHUNK,), jnp.int32),
            pltpu.VMEM((CHUNK, D), jnp.float32),
        ],
    )
    def deg_kernel(dst_hbm, zd_hbm, ones_hbm, deg_hbm, deg_sh, dst_v, ones_v):
        c = lax.axis_index("c")
        s = lax.axis_index("s")
        wid = c * NS + s
        acc_base = s * ROWS_PER_TEC
        out_base = c * N_ACC + s * ROWS_PER_TEC
        pieces = [(0, CHUNK), (CHUNK, CHUNK), (2 * CHUNK, CHUNK),
                  (3 * CHUNK, CHUNK), (4 * CHUNK, ROWS_PER_TEC - 4 * CHUNK)]

        pltpu.sync_copy(zd_hbm, ones_v)
        for off, sz in pieces:
            pltpu.sync_copy(ones_v.at[pl.ds(0, sz)],
                            deg_sh.at[pl.ds(acc_base + off, sz)])
        pltpu.sync_copy(ones_hbm, ones_v)
        plsc.subcore_barrier()

        @pl.loop(0, CHUNKS_PER_W)
        def chunk_body(j):
            base = wid * E_PER_W + j * CHUNK
            pltpu.sync_copy(dst_hbm.at[pl.ds(base, CHUNK)], dst_v)
            pltpu.sync_copy(ones_v, deg_sh.at[dst_v], add=True)

        plsc.subcore_barrier()
        for off, sz in pieces:
            pltpu.sync_copy(deg_sh.at[pl.ds(acc_base + off, sz)],
                            ones_v.at[pl.ds(0, sz)])
            pltpu.sync_copy(ones_v.at[pl.ds(0, sz)],
                            deg_hbm.at[pl.ds(out_base + off, sz)])

    return deg_kernel(dst1d, zrows, ones)


def _tc_combine(x, a0, a1, d0, d1, W_neigh, W_self, b2d, bias2d):
    R = 400
    grid = (N_NODES // R,)

    def body(x_ref, a0_ref, a1_ref, d0_ref, d1_ref, wn_ref, ws_ref,
             b_ref, bias_ref, o_ref):
        deg = d0_ref[:, :1] + d1_ref[:, :1]
        inv = 1.0 / jnp.maximum(deg, 1.0)
        mean = (a0_ref[...] + a1_ref[...]) * inv
        hn = lax.dot_general(mean, wn_ref[...], (((1,), (1,)), ((), ())),
                             preferred_element_type=jnp.float32)
        hs = lax.dot_general(x_ref[...], ws_ref[...], (((1,), (1,)), ((), ())),
                             preferred_element_type=jnp.float32)
        o_ref[...] = hs + hn + b_ref[...] + bias_ref[...]

    row_spec = pl.BlockSpec((R, D), lambda i: (i, 0))
    deg_spec = pl.BlockSpec((R, D), lambda i: (i, 0))
    full_spec = pl.BlockSpec((D, D), lambda i: (0, 0))
    vec_spec = pl.BlockSpec((1, D), lambda i: (0, 0))

    return pl.pallas_call(
        body,
        grid=grid,
        in_specs=[row_spec, row_spec, row_spec, deg_spec, deg_spec,
                  full_spec, full_spec, vec_spec, vec_spec],
        out_specs=row_spec,
        out_shape=jax.ShapeDtypeStruct((N_NODES, D), jnp.float32),
    )(x, a0, a1, d0, d1, W_neigh, W_self, b2d, bias2d)


def kernel(x, edge_index, W_neigh, W_self, b_self, bias):
    src = edge_index[0].astype(jnp.int32)
    dst = edge_index[1].astype(jnp.int32)
    n_pad = E_PAD - N_EDGES
    src2d = jnp.concatenate([src, jnp.zeros((n_pad,), jnp.int32)])
    dst2d = jnp.concatenate([dst, jnp.full((n_pad,), N_NODES, jnp.int32)])
    zrows = jnp.zeros((CHUNK, D), jnp.float32)
    ones = jnp.ones((CHUNK, D), jnp.float32)

    agg = _sc_aggregate(x, src2d, dst2d, zrows)
    deg = _sc_degree(dst2d, zrows, ones)

    a0 = agg[:N_NODES]
    a1 = agg[N_ACC:N_ACC + N_NODES]
    d0 = deg[:N_NODES]
    d1 = deg[N_ACC:N_ACC + N_NODES]
    b2d = b_self.reshape(1, D)
    bias2d = bias.reshape(1, D)
    return _tc_combine(x, a0, a1, d0, d1, W_neigh, W_self, b2d, bias2d)


# speedup vs baseline: 3.5576x; 1.1668x over previous
import functools

import jax
import jax.numpy as jnp
from jax import lax
from jax.experimental import pallas as pl
from jax.experimental.pallas import tpu as pltpu, tpu_sc as plsc

N_NODES = 10000
N_EDGES = 320000
D = 128

NC = 2
NS = 16
NW = NC * NS

CHUNK = 128
E_PER_W = 10240
E_PAD = NW * E_PER_W
CHUNKS_PER_W = E_PER_W // CHUNK
N_ACC = 10112
ROWS_PER_TEC = N_ACC // NS
DEG_W = 16
STAGE = 16


def _sc_aggregate(x, src2d, dst2d, zrows):
    mesh = plsc.VectorSubcoreMesh(
        core_axis_name="c", subcore_axis_name="s", num_cores=NC, num_subcores=NS
    )

    @functools.partial(
        pl.kernel,
        out_type=jax.ShapeDtypeStruct((NC * N_ACC, D), jnp.float32),
        mesh=mesh,
        scratch_types=[
            pltpu.VMEM_SHARED((N_ACC, D), jnp.float32),
            pltpu.VMEM((CHUNK,), jnp.int32),
            pltpu.VMEM((CHUNK,), jnp.int32),
            pltpu.VMEM((CHUNK,), jnp.int32),
            pltpu.VMEM((CHUNK,), jnp.int32),
            pltpu.VMEM((CHUNK, D), jnp.float32),
            pltpu.VMEM((CHUNK, D), jnp.float32),
            pltpu.SemaphoreType.DMA,
            pltpu.SemaphoreType.DMA,
        ],
    )
    def agg_kernel(x_hbm, src_hbm, dst_hbm, zr_hbm,
                   agg_hbm,
                   acc_sh, src_a, dst_a, src_b, dst_b, rows_a, rows_b,
                   sem_a, sem_b):
        c = lax.axis_index("c")
        s = lax.axis_index("s")
        wid = c * NS + s
        acc_base = s * ROWS_PER_TEC
        out_base = c * N_ACC + s * ROWS_PER_TEC
        pieces = [(0, CHUNK), (CHUNK, CHUNK), (2 * CHUNK, CHUNK),
                  (3 * CHUNK, CHUNK), (4 * CHUNK, ROWS_PER_TEC - 4 * CHUNK)]

        pltpu.sync_copy(zr_hbm, rows_a)
        for off, sz in pieces:
            pltpu.sync_copy(rows_a.at[pl.ds(0, sz)],
                            acc_sh.at[pl.ds(acc_base + off, sz)])
        plsc.subcore_barrier()

        ebase = wid * E_PER_W
        pltpu.sync_copy(src_hbm.at[pl.ds(ebase, CHUNK)], src_a)
        pltpu.sync_copy(dst_hbm.at[pl.ds(ebase, CHUNK)], dst_a)
        pltpu.async_copy(x_hbm.at[src_a], rows_a, sem_a)

        @pl.loop(0, CHUNKS_PER_W // 2)
        def chunk_body(g):
            b1 = ebase + (2 * g + 1) * CHUNK
            pltpu.sync_copy(src_hbm.at[pl.ds(b1, CHUNK)], src_b)
            pltpu.sync_copy(dst_hbm.at[pl.ds(b1, CHUNK)], dst_b)
            pltpu.async_copy(x_hbm.at[src_b], rows_b, sem_b)
            pltpu.make_async_copy(x_hbm.at[src_a], rows_a, sem_a).wait()
            pltpu.sync_copy(rows_a, acc_sh.at[dst_a], add=True)
            b2 = ebase + (2 * g + 2) * CHUNK
            pltpu.sync_copy(src_hbm.at[pl.ds(b2, CHUNK)], src_a)
            pltpu.sync_copy(dst_hbm.at[pl.ds(b2, CHUNK)], dst_a)
            pltpu.async_copy(x_hbm.at[src_a], rows_a, sem_a)
            pltpu.make_async_copy(x_hbm.at[src_b], rows_b, sem_b).wait()
            pltpu.sync_copy(rows_b, acc_sh.at[dst_b], add=True)

        pltpu.make_async_copy(x_hbm.at[src_a], rows_a, sem_a).wait()
        plsc.subcore_barrier()
        for off, sz in pieces:
            pltpu.sync_copy(acc_sh.at[pl.ds(acc_base + off, sz)],
                            rows_a.at[pl.ds(0, sz)])
            pltpu.sync_copy(rows_a.at[pl.ds(0, sz)],
                            agg_hbm.at[pl.ds(out_base + off, sz)])

    return agg_kernel(x, src2d, dst2d, zrows)


def _sc_degree(dst1d, zrows, ones):
    mesh = plsc.VectorSubcoreMesh(
        core_axis_name="c", subcore_axis_name="s", num_cores=NC, num_subcores=NS
    )

    @functools.partial(
        pl.kernel,
        out_type=jax.ShapeDtypeStruct((NC * N_ACC, D), jnp.float32),
        mesh=mesh,
        scratch_types=[
            pltpu.VMEM_SHARED((N_ACC, D), jnp.float32),
            pltpu.VMEM((CHUNK,), jnp.int32),
            pltpu.VMEM((CHUNK, D), jnp.float32),
        ],
    )
    def deg_kernel(dst_hbm, zd_hbm, ones_hbm, deg_hbm, deg_sh, dst_v, ones_v):
        c = lax.axis_index("c")
        s = lax.axis_index("s")
        wid = c * NS + s
        acc_base = s * ROWS_PER_TEC
        out_base = c * N_ACC + s * ROWS_PER_TEC
        pieces = [(0, CHUNK), (CHUNK, CHUNK), (2 * CHUNK, CHUNK),
                  (3 * CHUNK, CHUNK), (4 * CHUNK, ROWS_PER_TEC - 4 * CHUNK)]

        pltpu.sync_copy(zd_hbm, ones_v)
        for off, sz in pieces:
            pltpu.sync_copy(ones_v.at[pl.ds(0, sz)],
                            deg_sh.at[pl.ds(acc_base + off, sz)])
        pltpu.sync_copy(ones_hbm, ones_v)
        plsc.subcore_barrier()

        @pl.loop(0, CHUNKS_PER_W)
        def chunk_body(j):
            base = wid * E_PER_W + j * CHUNK
            pltpu.sync_copy(dst_hbm.at[pl.ds(base, CHUNK)], dst_v)
            pltpu.sync_copy(ones_v, deg_sh.at[dst_v], add=True)

        plsc.subcore_barrier()
        for off, sz in pieces:
            pltpu.sync_copy(deg_sh.at[pl.ds(acc_base + off, sz)],
                            ones_v.at[pl.ds(0, sz)])
            pltpu.sync_copy(ones_v.at[pl.ds(0, sz)],
                            deg_hbm.at[pl.ds(out_base + off, sz)])

    return deg_kernel(dst1d, zrows, ones)


def _tc_combine(x, a0, a1, d0, d1, W_neigh, W_self, b2d, bias2d):
    R = 400
    grid = (N_NODES // R,)

    def body(x_ref, a0_ref, a1_ref, d0_ref, d1_ref, wn_ref, ws_ref,
             b_ref, bias_ref, o_ref):
        deg = d0_ref[:, :1] + d1_ref[:, :1]
        inv = 1.0 / jnp.maximum(deg, 1.0)
        mean = (a0_ref[...] + a1_ref[...]) * inv
        hn = lax.dot_general(mean, wn_ref[...], (((1,), (1,)), ((), ())),
                             preferred_element_type=jnp.float32)
        hs = lax.dot_general(x_ref[...], ws_ref[...], (((1,), (1,)), ((), ())),
                             preferred_element_type=jnp.float32)
        o_ref[...] = hs + hn + b_ref[...] + bias_ref[...]

    row_spec = pl.BlockSpec((R, D), lambda i: (i, 0))
    deg_spec = pl.BlockSpec((R, D), lambda i: (i, 0))
    full_spec = pl.BlockSpec((D, D), lambda i: (0, 0))
    vec_spec = pl.BlockSpec((1, D), lambda i: (0, 0))

    return pl.pallas_call(
        body,
        grid=grid,
        in_specs=[row_spec, row_spec, row_spec, deg_spec, deg_spec,
                  full_spec, full_spec, vec_spec, vec_spec],
        out_specs=row_spec,
        out_shape=jax.ShapeDtypeStruct((N_NODES, D), jnp.float32),
    )(x, a0, a1, d0, d1, W_neigh, W_self, b2d, bias2d)


def kernel(x, edge_index, W_neigh, W_self, b_self, bias):
    src = edge_index[0].astype(jnp.int32)
    dst = edge_index[1].astype(jnp.int32)
    n_pad = E_PAD - N_EDGES + CHUNK
    src2d = jnp.concatenate([src, jnp.zeros((n_pad,), jnp.int32)])
    dst2d = jnp.concatenate([dst, jnp.full((n_pad,), N_NODES, jnp.int32)])
    zrows = jnp.zeros((CHUNK, D), jnp.float32)
    ones = jnp.ones((CHUNK, D), jnp.float32)

    agg = _sc_aggregate(x, src2d, dst2d, zrows)
    deg = _sc_degree(dst2d, zrows, ones)

    a0 = agg[:N_NODES]
    a1 = agg[N_ACC:N_ACC + N_NODES]
    d0 = deg[:N_NODES]
    d1 = deg[N_ACC:N_ACC + N_NODES]
    b2d = b_self.reshape(1, D)
    bias2d = bias.reshape(1, D)
    return _tc_combine(x, a0, a1, d0, d1, W_neigh, W_self, b2d, bias2d)
